# baseline (speedup 1.0000x reference)
import numpy as np
import jax
import jax.numpy as jnp
from jax.experimental import pallas as pl
from jax.experimental.pallas import tpu as pltpu

_BS, _T, _Q, _C = 8, 32, 8, 256
_N = _BS * _Q
_TOPK = 5

_BP = [25, 30, 28, 13, 22, 14, 30, 29, 12, 13, 13, 2, 25, 20, 20, 27,
       24, 13, 10, 18, 11, 26, 27, 17, 14, 17, 18, 18, 15, 5, 2, 20,
       22, 14, 17, 11, 28, 22, 6, 17, 25, 15, 27, 26, 2, 18, 10, 26,
       19, 24, 13, 23, 18, 5, 18, 16, 30, 21, 22, 19, 24, 30, 7, 8]
_USED_T = sorted(set(_BP))


def _build_consts():
    bp_i = np.asarray(_BP, dtype=np.int64)
    bp = bp_i.astype(np.float32)
    bh = np.float32(0.0)
    bt = np.float32(_T - 1)
    alpha = (bp - bh) / (bt - bh)
    sigma = alpha * (bt - bp)
    inv2s2 = (1.0 / (2.0 * sigma * sigma)).astype(np.float32)
    oh = np.zeros((_T, _N, 1), np.float32)
    oh[bp_i, np.arange(_N), 0] = 1.0
    a1m = (1.0 - alpha).astype(np.float32).reshape(_N, 1)
    aa = alpha.astype(np.float32).reshape(_N, 1)
    return oh, a1m, aa, inv2s2.reshape(_N, 1)


_OH, _A1M, _AA, _IS2 = _build_consts()
_C11 = (((1,), (1,)), ((), ()))
_C10 = (((1,), (0,)), ((), ()))


def _body(x_ref, w_ref, b_ref, oh_ref, a1m_ref, aa_ref, is2_ref,
          o1_ref, o2_ref, cur_ref):
    g = pl.program_id(0)
    ones_c = jnp.ones((_C, 1), jnp.float32)

    xg = x_ref[...].reshape(_T * _Q, _C)
    fe = jax.lax.dot_general(xg, w_ref[...], _C11,
                             preferred_element_type=jnp.float32)
    fe = fe + b_ref[...]
    ss = jax.lax.dot_general(fe * fe, ones_c, _C10,
                             preferred_element_type=jnp.float32)
    inv = jax.lax.rsqrt(jnp.maximum(ss, 1e-24))
    cur_ref[g] = (fe * inv).reshape(_T, _Q, _C)

    @pl.when(g == _BS - 1)
    def _tail():
        cur4 = cur_ref[...]

        def tslice(t):
            return jnp.reshape(
                jax.lax.slice(cur4, (0, t, 0, 0), (_BS, t + 1, _Q, _C)),
                (_N, _C))

        t0 = tslice(0)
        t31 = tslice(_T - 1)
        base = a1m_ref[...] * t0 + aa_ref[...] * t31
        bnsq = jax.lax.dot_general(base * base, ones_c, _C10,
                                   preferred_element_type=jnp.float32)

        oh = oh_ref[...]
        parts = [jnp.zeros((_N, _N), jnp.float32) for _ in range(4)]
        for k, t in enumerate(_USED_T):
            dt = jax.lax.dot_general(base, tslice(t), _C11,
                                     preferred_element_type=jnp.float32)
            wt = jnp.reshape(jax.lax.slice(oh, (t, 0, 0), (t + 1, _N, 1)),
                             (_N, 1))
            parts[k % 4] = parts[k % 4] + wt * dt
        d = (parts[0] + parts[1]) + (parts[2] + parts[3])

        dist = (2.0 * d - (1.0 + bnsq)) * is2_ref[...]

        rows = jax.lax.broadcasted_iota(jnp.int32, (_N, _N), 0)
        cols = jax.lax.broadcasted_iota(jnp.int32, (_N, _N), 1)
        eye = rows == cols
        ones_n = jnp.ones((_N, 1), jnp.float32)
        self_d = jax.lax.dot_general(jnp.where(eye, dist, 0.0), ones_n,
                                     _C10,
                                     preferred_element_type=jnp.float32)
        dm = jnp.where(eye, -1e30, dist)

        numer = jnp.exp(self_d)
        acc = numer
        for _ in range(_TOPK):
            m = jnp.max(dm, axis=1, keepdims=True)
            acc = acc + jnp.exp(m)
            cand = jnp.where(dm >= m, cols, _N)
            amin = jnp.min(cand, axis=1, keepdims=True)
            dm = jnp.where(cols == amin, -1e30, dm)

        score = jax.lax.dot_general(t0 * t31, ones_c, _C10,
                                    preferred_element_type=jnp.float32)
        z = 0.3 - score
        sp = jnp.maximum(z, 0.0) + jnp.log1p(jnp.exp(-jnp.abs(z)))
        o1_ref[...] = jnp.reshape(jnp.sum(numer / acc) * (1.0 / _N), (1, 1))
        o2_ref[...] = jnp.reshape(jnp.sum(sp) * (1.0 / _N), (1, 1))


def kernel(frame_embeds, W, b):
    o1, o2 = pl.pallas_call(
        _body,
        grid=(_BS,),
        in_specs=[
            pl.BlockSpec((1, _T, _Q, _C), lambda g: (g, 0, 0, 0)),
            pl.BlockSpec((_C, _C), lambda g: (0, 0)),
            pl.BlockSpec((1, _C), lambda g: (0, 0)),
            pl.BlockSpec((_T, _N, 1), lambda g: (0, 0, 0)),
            pl.BlockSpec((_N, 1), lambda g: (0, 0)),
            pl.BlockSpec((_N, 1), lambda g: (0, 0)),
            pl.BlockSpec((_N, 1), lambda g: (0, 0)),
        ],
        out_specs=[
            pl.BlockSpec((1, 1), lambda g: (0, 0)),
            pl.BlockSpec((1, 1), lambda g: (0, 0)),
        ],
        out_shape=[
            jax.ShapeDtypeStruct((1, 1), jnp.float32),
            jax.ShapeDtypeStruct((1, 1), jnp.float32),
        ],
        scratch_shapes=[pltpu.VMEM((_BS, _T, _Q, _C), jnp.float32)],
    )(frame_embeds, W, b.reshape(1, _C), _OH, _A1M, _AA, _IS2)
    return o1[0, 0], o2[0, 0]


# speedup vs baseline: 1.0413x; 1.0413x over previous
import numpy as np
import jax
import jax.numpy as jnp
from jax.experimental import pallas as pl
from jax.experimental.pallas import tpu as pltpu

_BS, _T, _Q, _C = 8, 32, 8, 256
_N = _BS * _Q
_TOPK = 5

_BP = [25, 30, 28, 13, 22, 14, 30, 29, 12, 13, 13, 2, 25, 20, 20, 27,
       24, 13, 10, 18, 11, 26, 27, 17, 14, 17, 18, 18, 15, 5, 2, 20,
       22, 14, 17, 11, 28, 22, 6, 17, 25, 15, 27, 26, 2, 18, 10, 26,
       19, 24, 13, 23, 18, 5, 18, 16, 30, 21, 22, 19, 24, 30, 7, 8]
_USED_T = sorted(set(_BP))


def _build_consts():
    bp_i = np.asarray(_BP, dtype=np.int64)
    bp = bp_i.astype(np.float32)
    bh = np.float32(0.0)
    bt = np.float32(_T - 1)
    alpha = (bp - bh) / (bt - bh)
    sigma = alpha * (bt - bp)
    inv2s2 = (1.0 / (2.0 * sigma * sigma)).astype(np.float32)
    oh = np.zeros((_T, _N, 1), np.float32)
    oh[bp_i, np.arange(_N), 0] = 1.0
    a1m = (1.0 - alpha).astype(np.float32).reshape(_N, 1)
    aa = alpha.astype(np.float32).reshape(_N, 1)
    return oh, a1m, aa, inv2s2.reshape(_N, 1)


_OH, _A1M, _AA, _IS2 = _build_consts()
_C11 = (((1,), (1,)), ((), ()))
_C10 = (((1,), (0,)), ((), ()))


def _body(x_hbm, w_ref, b_ref, oh_ref, a1m_ref, aa_ref, is2_ref,
          o1_ref, o2_ref, xbuf_ref, cur_ref, sems):
    ones_c = jnp.ones((_C, 1), jnp.float32)

    copies = [
        pltpu.make_async_copy(x_hbm.at[pl.ds(g, 1)], xbuf_ref.at[pl.ds(g, 1)],
                              sems.at[g])
        for g in range(_BS)
    ]
    for c in copies:
        c.start()
    w = w_ref[...]
    for g in range(_BS):
        copies[g].wait()
        xg = xbuf_ref[g].reshape(_T * _Q, _C)
        fe = jax.lax.dot_general(xg, w, _C11,
                                 preferred_element_type=jnp.float32)
        fe = fe + b_ref[...]
        ss = jax.lax.dot_general(fe * fe, ones_c, _C10,
                                 preferred_element_type=jnp.float32)
        inv = jax.lax.rsqrt(jnp.maximum(ss, 1e-24))
        cur_ref[g] = (fe * inv).reshape(_T, _Q, _C)

    if True:
        cur4 = cur_ref[...]

        def tslice(t):
            return jnp.reshape(
                jax.lax.slice(cur4, (0, t, 0, 0), (_BS, t + 1, _Q, _C)),
                (_N, _C))

        t0 = tslice(0)
        t31 = tslice(_T - 1)
        base = a1m_ref[...] * t0 + aa_ref[...] * t31
        bnsq = jax.lax.dot_general(base * base, ones_c, _C10,
                                   preferred_element_type=jnp.float32)

        oh = oh_ref[...]
        parts = [jnp.zeros((_N, _N), jnp.float32) for _ in range(4)]
        for k, t in enumerate(_USED_T):
            dt = jax.lax.dot_general(base, tslice(t), _C11,
                                     preferred_element_type=jnp.float32)
            wt = jnp.reshape(jax.lax.slice(oh, (t, 0, 0), (t + 1, _N, 1)),
                             (_N, 1))
            parts[k % 4] = parts[k % 4] + wt * dt
        d = (parts[0] + parts[1]) + (parts[2] + parts[3])

        dist = (2.0 * d - (1.0 + bnsq)) * is2_ref[...]

        rows = jax.lax.broadcasted_iota(jnp.int32, (_N, _N), 0)
        cols = jax.lax.broadcasted_iota(jnp.int32, (_N, _N), 1)
        eye = rows == cols
        ones_n = jnp.ones((_N, 1), jnp.float32)
        self_d = jax.lax.dot_general(jnp.where(eye, dist, 0.0), ones_n,
                                     _C10,
                                     preferred_element_type=jnp.float32)
        dm = jnp.where(eye, -1e30, dist)

        numer = jnp.exp(self_d)
        acc = numer
        for _ in range(_TOPK):
            m = jnp.max(dm, axis=1, keepdims=True)
            acc = acc + jnp.exp(m)
            cand = jnp.where(dm >= m, cols, _N)
            amin = jnp.min(cand, axis=1, keepdims=True)
            dm = jnp.where(cols == amin, -1e30, dm)

        score = jax.lax.dot_general(t0 * t31, ones_c, _C10,
                                    preferred_element_type=jnp.float32)
        z = 0.3 - score
        sp = jnp.maximum(z, 0.0) + jnp.log1p(jnp.exp(-jnp.abs(z)))
        o1_ref[...] = jnp.reshape(jnp.sum(numer / acc) * (1.0 / _N), (1, 1))
        o2_ref[...] = jnp.reshape(jnp.sum(sp) * (1.0 / _N), (1, 1))


def kernel(frame_embeds, W, b):
    o1, o2 = pl.pallas_call(
        _body,
        in_specs=[
            pl.BlockSpec(memory_space=pl.ANY),
            pl.BlockSpec((_C, _C), lambda: (0, 0)),
            pl.BlockSpec((1, _C), lambda: (0, 0)),
            pl.BlockSpec((_T, _N, 1), lambda: (0, 0, 0)),
            pl.BlockSpec((_N, 1), lambda: (0, 0)),
            pl.BlockSpec((_N, 1), lambda: (0, 0)),
            pl.BlockSpec((_N, 1), lambda: (0, 0)),
        ],
        out_shape=[
            jax.ShapeDtypeStruct((1, 1), jnp.float32),
            jax.ShapeDtypeStruct((1, 1), jnp.float32),
        ],
        scratch_shapes=[
            pltpu.VMEM((_BS, _T, _Q, _C), jnp.float32),
            pltpu.VMEM((_BS, _T, _Q, _C), jnp.float32),
            pltpu.SemaphoreType.DMA((_BS,)),
        ],
    )(frame_embeds, W, b.reshape(1, _C), _OH, _A1M, _AA, _IS2)
    return o1[0, 0], o2[0, 0]


# speedup vs baseline: 1.6893x; 1.6223x over previous
import numpy as np
import jax
import jax.numpy as jnp
from jax.experimental import pallas as pl
from jax.experimental.pallas import tpu as pltpu

_BS, _T, _Q, _C = 8, 32, 8, 256
_N = _BS * _Q
_TOPK = 5

_BP = [25, 30, 28, 13, 22, 14, 30, 29, 12, 13, 13, 2, 25, 20, 20, 27,
       24, 13, 10, 18, 11, 26, 27, 17, 14, 17, 18, 18, 15, 5, 2, 20,
       22, 14, 17, 11, 28, 22, 6, 17, 25, 15, 27, 26, 2, 18, 10, 26,
       19, 24, 13, 23, 18, 5, 18, 16, 30, 21, 22, 19, 24, 30, 7, 8]
_USED_T = sorted(set(_BP))


def _build_consts():
    bp_i = np.asarray(_BP, dtype=np.int64)
    bp = bp_i.astype(np.float32)
    bh = np.float32(0.0)
    bt = np.float32(_T - 1)
    alpha = (bp - bh) / (bt - bh)
    sigma = alpha * (bt - bp)
    inv2s2 = (1.0 / (2.0 * sigma * sigma)).astype(np.float32)
    oh = np.zeros((_T, _N, 1), np.float32)
    oh[bp_i, np.arange(_N), 0] = 1.0
    a1m = (1.0 - alpha).astype(np.float32).reshape(_N, 1)
    aa = alpha.astype(np.float32).reshape(_N, 1)
    return oh, a1m, aa, inv2s2.reshape(_N, 1)


_OH, _A1M, _AA, _IS2 = _build_consts()
_C11 = (((1,), (1,)), ((), ()))
_C10 = (((1,), (0,)), ((), ()))


def _body(x_ref, w_ref, b_ref, oh_ref, a1m_ref, aa_ref, is2_ref,
          o1_ref, o2_ref):
    x = x_ref[...]
    w = w_ref[...]
    fe = jax.lax.dot_general(x, w, _C11,
                             preferred_element_type=jnp.float32)
    fe = fe + b_ref[...]
    ones_c = jnp.ones((_C, 1), jnp.float32)
    ss = jax.lax.dot_general(fe * fe, ones_c, _C10,
                             preferred_element_type=jnp.float32)
    inv = jax.lax.rsqrt(jnp.maximum(ss, 1e-24))
    fe4 = fe.reshape(_BS, _T, _Q, _C)
    inv4 = inv.reshape(_BS, _T, _Q, 1)

    def tslice(t):
        ft = jnp.reshape(
            jax.lax.slice(fe4, (0, t, 0, 0), (_BS, t + 1, _Q, _C)),
            (_N, _C))
        it = jnp.reshape(
            jax.lax.slice(inv4, (0, t, 0, 0), (_BS, t + 1, _Q, 1)),
            (_N, 1))
        return ft * it

    t0 = tslice(0)
    t31 = tslice(_T - 1)
    base = a1m_ref[...] * t0 + aa_ref[...] * t31
    bnsq = jax.lax.dot_general(base * base, ones_c, _C10,
                               preferred_element_type=jnp.float32)

    oh = oh_ref[...]
    parts = [jnp.zeros((_N, _N), jnp.float32) for _ in range(4)]
    for k, t in enumerate(_USED_T):
        dt = jax.lax.dot_general(base, tslice(t), _C11,
                                 preferred_element_type=jnp.float32)
        wt = jnp.reshape(jax.lax.slice(oh, (t, 0, 0), (t + 1, _N, 1)),
                         (_N, 1))
        parts[k % 4] = parts[k % 4] + wt * dt
    d = (parts[0] + parts[1]) + (parts[2] + parts[3])

    dist = (2.0 * d - (1.0 + bnsq)) * is2_ref[...]

    rows = jax.lax.broadcasted_iota(jnp.int32, (_N, _N), 0)
    cols = jax.lax.broadcasted_iota(jnp.int32, (_N, _N), 1)
    eye = rows == cols
    ones_n = jnp.ones((_N, 1), jnp.float32)
    self_d = jax.lax.dot_general(jnp.where(eye, dist, 0.0), ones_n,
                                 _C10,
                                 preferred_element_type=jnp.float32)
    dm = jnp.where(eye, -1e30, dist)

    numer = jnp.exp(self_d)
    acc = numer
    for _ in range(_TOPK):
        m = jnp.max(dm, axis=1, keepdims=True)
        acc = acc + jnp.exp(m)
        cand = jnp.where(dm >= m, cols, _N)
        amin = jnp.min(cand, axis=1, keepdims=True)
        dm = jnp.where(cols == amin, -1e30, dm)

    score = jax.lax.dot_general(t0 * t31, ones_c, _C10,
                                preferred_element_type=jnp.float32)
    z = 0.3 - score
    sp = jnp.maximum(z, 0.0) + jnp.log1p(jnp.exp(-jnp.abs(z)))
    o1_ref[...] = jnp.reshape(jnp.sum(numer / acc) * (1.0 / _N), (1, 1))
    o2_ref[...] = jnp.reshape(jnp.sum(sp) * (1.0 / _N), (1, 1))


def kernel(frame_embeds, W, b):
    x2 = frame_embeds.reshape(_BS * _T * _Q, _C)
    o1, o2 = pl.pallas_call(
        _body,
        out_shape=[
            jax.ShapeDtypeStruct((1, 1), jnp.float32),
            jax.ShapeDtypeStruct((1, 1), jnp.float32),
        ],
    )(x2, W, b.reshape(1, _C), _OH, _A1M, _AA, _IS2)
    return o1[0, 0], o2[0, 0]
